# baseline (device time: 87541 ns/iter reference)
import jax
import jax.numpy as jnp
from jax import lax
from jax.experimental import pallas as pl
from jax.experimental.pallas import tpu as pltpu

N_DEV = 8
B, SQ, SKV, HQ, DH, DM = 2, 512, 512, 64, 64, 768
H_PER = HQ // N_DEV
D_PER = H_PER * DH
CHUNK = SQ // N_DEV
N_STEPS = 2 * (N_DEV - 1)
WINDOW = 128


def kernel(x, Wq, K_ext, V_ext, Wo):
    my = lax.axis_index("i")

    K = lax.dynamic_slice_in_dim(K_ext, my * H_PER, H_PER, axis=2)
    V = lax.dynamic_slice_in_dim(V_ext, my * H_PER, H_PER, axis=2)
    K = jnp.transpose(K, (2, 0, 1, 3)).astype(jnp.bfloat16)
    V = jnp.transpose(V, (2, 0, 1, 3)).astype(jnp.bfloat16)
    Wq_r = jnp.transpose(
        Wq.reshape(DM, H_PER, DH), (1, 0, 2)
    ).astype(jnp.bfloat16)
    Wo_r = Wo.reshape(H_PER, DH, DM).astype(jnp.bfloat16)
    x_b = x.astype(jnp.bfloat16)

    def body(x_ref, wq_ref, k_ref, v_ref, wo_ref, out_ref,
             send_buf, recv_buf, send_sems, recv_sems):
        my_i = lax.axis_index("i")
        left = (my_i - 1) % N_DEV
        right = (my_i + 1) % N_DEV

        barrier = pltpu.get_barrier_semaphore()
        for nbr in (left, right):
            pl.semaphore_signal(
                barrier, inc=1,
                device_id=(nbr,), device_id_type=pl.DeviceIdType.MESH,
            )
        pl.semaphore_wait(barrier, 2)

        for b in range(B):
            xb = x_ref[b]
            acc = jnp.zeros((SQ, DM), jnp.float32)
            for h in range(H_PER):
                q = jnp.dot(xb, wq_ref[h],
                            preferred_element_type=jnp.float32)
                s = lax.dot_general(
                    q.astype(jnp.bfloat16), k_ref[h, b],
                    (((1,), (1,)), ((), ())),
                    preferred_element_type=jnp.float32,
                ) * 0.125
                qi = lax.broadcasted_iota(jnp.int32, (SQ, SKV), 0)
                ki = lax.broadcasted_iota(jnp.int32, (SQ, SKV), 1)
                mask = jnp.abs(qi - ki) <= WINDOW
                s = jnp.where(mask, s, -1e9)
                m = jnp.max(s, axis=-1, keepdims=True)
                w = jnp.exp(s - m)
                w = w / jnp.sum(w, axis=-1, keepdims=True)
                ctx = jnp.dot(w.astype(jnp.bfloat16), v_ref[h, b],
                              preferred_element_type=jnp.float32)
                acc = acc + jnp.dot(ctx.astype(jnp.bfloat16), wo_ref[h],
                                    preferred_element_type=jnp.float32)
            out_ref[b] = acc

        def hop(step, send_idx, recv_idx, reduce):
            send_buf[step] = out_ref[
                :, pl.ds(send_idx * CHUNK, CHUNK), :
            ].astype(jnp.bfloat16)
            rdma = pltpu.make_async_remote_copy(
                src_ref=send_buf.at[step],
                dst_ref=recv_buf.at[step],
                send_sem=send_sems.at[step],
                recv_sem=recv_sems.at[step],
                device_id=(right,),
                device_id_type=pl.DeviceIdType.MESH,
            )
            rdma.start()
            rdma.wait()
            got = recv_buf[step].astype(jnp.float32)
            if reduce:
                got = got + out_ref[:, pl.ds(recv_idx * CHUNK, CHUNK), :]
            out_ref[:, pl.ds(recv_idx * CHUNK, CHUNK), :] = got

        for s_i in range(N_DEV - 1):
            hop(s_i, (my_i - s_i) % N_DEV, (my_i - s_i - 1) % N_DEV, True)
        for t in range(N_DEV - 1):
            hop(N_DEV - 1 + t, (my_i + 1 - t) % N_DEV, (my_i - t) % N_DEV,
                False)

    return pl.pallas_call(
        body,
        out_shape=jax.ShapeDtypeStruct((B, SQ, DM), jnp.float32),
        in_specs=[pl.BlockSpec(memory_space=pltpu.VMEM)] * 5,
        out_specs=pl.BlockSpec(memory_space=pltpu.VMEM),
        scratch_shapes=[
            pltpu.VMEM((N_STEPS, B, CHUNK, DM), jnp.bfloat16),
            pltpu.VMEM((N_STEPS, B, CHUNK, DM), jnp.bfloat16),
            pltpu.SemaphoreType.DMA((N_STEPS,)),
            pltpu.SemaphoreType.DMA((N_STEPS,)),
        ],
        compiler_params=pltpu.CompilerParams(collective_id=0),
    )(x_b, Wq_r, K, V, Wo_r)


# device time: 73964 ns/iter; 1.1836x vs baseline; 1.1836x over previous
import jax
import jax.numpy as jnp
from jax import lax
from jax.experimental import pallas as pl
from jax.experimental.pallas import tpu as pltpu

N_DEV = 8
B, SQ, SKV, HQ, DH, DM = 2, 512, 512, 64, 64, 768
H_PER = HQ // N_DEV
D_PER = H_PER * DH
CHUNK = SQ // N_DEV
N_STEPS = 2 * (N_DEV - 1)
WINDOW = 128


def kernel(x, Wq, K_ext, V_ext, Wo):
    my = lax.axis_index("i")

    K = lax.dynamic_slice_in_dim(K_ext, my * H_PER, H_PER, axis=2)
    V = lax.dynamic_slice_in_dim(V_ext, my * H_PER, H_PER, axis=2)
    K = jnp.transpose(K, (2, 0, 1, 3)).astype(jnp.bfloat16)
    V = jnp.transpose(V, (2, 0, 1, 3)).astype(jnp.bfloat16)
    Wq_r = jnp.transpose(
        Wq.reshape(DM, H_PER, DH), (1, 0, 2)
    ).astype(jnp.bfloat16)
    Wo_r = Wo.reshape(H_PER, DH, DM).astype(jnp.bfloat16)
    x_b = x.astype(jnp.bfloat16)

    def body(x_ref, wq_ref, k_ref, v_ref, wo_ref, out_ref,
             send_buf, recv_buf, send_sems, recv_sems):
        my_i = lax.axis_index("i")
        i0 = my_i % 2
        i1 = (my_i // 2) % 2
        i2 = (my_i // 4) % 2
        p1 = my_i ^ 1
        p2 = my_i ^ 2
        p4 = my_i ^ 4

        barrier = pltpu.get_barrier_semaphore()
        for nbr in (p1, p2, p4):
            pl.semaphore_signal(
                barrier, inc=1,
                device_id=(nbr,), device_id_type=pl.DeviceIdType.MESH,
            )
        pl.semaphore_wait(barrier, 3)

        for b in range(B):
            xb = x_ref[b]
            acc = jnp.zeros((SQ, DM), jnp.float32)
            for h in range(H_PER):
                q = jnp.dot(xb, wq_ref[h],
                            preferred_element_type=jnp.float32)
                s = lax.dot_general(
                    q.astype(jnp.bfloat16), k_ref[h, b],
                    (((1,), (1,)), ((), ())),
                    preferred_element_type=jnp.float32,
                ) * 0.125
                qi = lax.broadcasted_iota(jnp.int32, (SQ, SKV), 0)
                ki = lax.broadcasted_iota(jnp.int32, (SQ, SKV), 1)
                mask = jnp.abs(qi - ki) <= WINDOW
                s = jnp.where(mask, s, -1e9)
                m = jnp.max(s, axis=-1, keepdims=True)
                w = jnp.exp(s - m)
                w = w / jnp.sum(w, axis=-1, keepdims=True)
                ctx = jnp.dot(w.astype(jnp.bfloat16), v_ref[h, b],
                              preferred_element_type=jnp.float32)
                acc = acc + jnp.dot(ctx.astype(jnp.bfloat16), wo_ref[h],
                                    preferred_element_type=jnp.float32)
            out_ref[b] = acc

        slot = [0]

        def xfer(partner, send_chunks, recv_chunks, reduce):
            base = slot[0]
            rdmas = []
            for j, c in enumerate(send_chunks):
                s = base + j
                send_buf[s] = out_ref[
                    :, pl.ds(c * CHUNK, CHUNK), :
                ].astype(jnp.bfloat16)
                rdma = pltpu.make_async_remote_copy(
                    src_ref=send_buf.at[s],
                    dst_ref=recv_buf.at[s],
                    send_sem=send_sems.at[s],
                    recv_sem=recv_sems.at[s],
                    device_id=(partner,),
                    device_id_type=pl.DeviceIdType.MESH,
                )
                rdma.start()
                rdmas.append(rdma)
            for j, c in enumerate(recv_chunks):
                rdmas[j].wait()
                got = recv_buf[base + j].astype(jnp.float32)
                if reduce:
                    got = got + out_ref[:, pl.ds(c * CHUNK, CHUNK), :]
                out_ref[:, pl.ds(c * CHUNK, CHUNK), :] = got
            slot[0] = base + len(send_chunks)

        ab = [(0, 0), (0, 1), (1, 0), (1, 1)]
        xfer(p1, [4 * a + 2 * b_ + (1 - i0) for a, b_ in ab],
                 [4 * a + 2 * b_ + i0 for a, b_ in ab], True)
        xfer(p4, [4 * (1 - i2) + 2 * b_ + i0 for b_ in (0, 1)],
                 [4 * i2 + 2 * b_ + i0 for b_ in (0, 1)], True)
        xfer(p2, [4 * i2 + 2 * (1 - i1) + i0],
                 [4 * i2 + 2 * i1 + i0], True)
        xfer(p2, [4 * i2 + 2 * i1 + i0],
                 [4 * i2 + 2 * (1 - i1) + i0], False)
        xfer(p4, [4 * i2 + 2 * b_ + i0 for b_ in (0, 1)],
                 [4 * (1 - i2) + 2 * b_ + i0 for b_ in (0, 1)], False)
        xfer(p1, [4 * a + 2 * b_ + i0 for a, b_ in ab],
                 [4 * a + 2 * b_ + (1 - i0) for a, b_ in ab], False)

    return pl.pallas_call(
        body,
        out_shape=jax.ShapeDtypeStruct((B, SQ, DM), jnp.float32),
        in_specs=[pl.BlockSpec(memory_space=pltpu.VMEM)] * 5,
        out_specs=pl.BlockSpec(memory_space=pltpu.VMEM),
        scratch_shapes=[
            pltpu.VMEM((N_STEPS, B, CHUNK, DM), jnp.bfloat16),
            pltpu.VMEM((N_STEPS, B, CHUNK, DM), jnp.bfloat16),
            pltpu.SemaphoreType.DMA((N_STEPS,)),
            pltpu.SemaphoreType.DMA((N_STEPS,)),
        ],
        compiler_params=pltpu.CompilerParams(collective_id=0),
    )(x_b, Wq_r, K, V, Wo_r)


# device time: 66573 ns/iter; 1.3150x vs baseline; 1.1110x over previous
import jax
import jax.numpy as jnp
from jax import lax
from jax.experimental import pallas as pl
from jax.experimental.pallas import tpu as pltpu

N_DEV = 8
B, SQ, SKV, HQ, DH, DM = 2, 512, 512, 64, 64, 768
H_PER = HQ // N_DEV
HALF = SQ // 2
WINDOW = 128
_COMM = True


def kernel(x, Wq, K_ext, V_ext, Wo):
    my = lax.axis_index("i")

    K = lax.dynamic_slice_in_dim(K_ext, my * H_PER, H_PER, axis=2)
    V = lax.dynamic_slice_in_dim(V_ext, my * H_PER, H_PER, axis=2)
    K = jnp.transpose(K, (2, 0, 1, 3)).astype(jnp.bfloat16)
    V = jnp.transpose(V, (2, 0, 1, 3)).astype(jnp.bfloat16)
    Wq_r = jnp.transpose(
        Wq.reshape(DM, H_PER, DH), (1, 0, 2)
    ).astype(jnp.bfloat16)
    Wo_r = Wo.reshape(H_PER, DH, DM).astype(jnp.bfloat16)
    x_b = x.astype(jnp.bfloat16)

    def body(x_ref, wq_ref, k_ref, v_ref, wo_ref, out_ref,
             rbuf1, rbuf2, rbuf3, send_sems, recv_sems):
        my_i = lax.axis_index("i")
        i0 = my_i % 2
        i1 = (my_i // 2) % 2
        i2 = (my_i // 4) % 2
        p1 = my_i ^ 1
        p2 = my_i ^ 2
        p4 = my_i ^ 4

        if _COMM:
            barrier = pltpu.get_barrier_semaphore()
            for nbr in (p1, p2, p4):
                pl.semaphore_signal(
                    barrier, inc=1,
                    device_id=(nbr,), device_id_type=pl.DeviceIdType.MESH,
                )
            pl.semaphore_wait(barrier, 3)

        rr = lax.broadcasted_iota(jnp.int32, (HALF, SKV), 0)
        cc = lax.broadcasted_iota(jnp.int32, (HALF, SKV), 1)

        def compute_half(base):
            bias = jnp.where(
                jnp.abs(rr + base - cc) <= WINDOW, 0.0, -1e9
            ).astype(jnp.float32)
            for b in range(B):
                xb = x_ref[b, pl.ds(base, HALF), :]
                acc = jnp.zeros((HALF, DM), jnp.float32)
                for h in range(H_PER):
                    q = jnp.dot(xb, wq_ref[h],
                                preferred_element_type=jnp.float32
                                ).astype(jnp.bfloat16)
                    s = lax.dot_general(
                        q, k_ref[h, b], (((1,), (1,)), ((), ())),
                        preferred_element_type=jnp.float32,
                    ) * 0.125 + bias
                    w = jnp.exp(s)
                    w = (w / jnp.sum(w, axis=-1, keepdims=True)
                         ).astype(jnp.bfloat16)
                    ctx = jnp.dot(w, v_ref[h, b],
                                  preferred_element_type=jnp.float32
                                  ).astype(jnp.bfloat16)
                    acc = acc + jnp.dot(ctx, wo_ref[h],
                                        preferred_element_type=jnp.float32)
                out_ref[b, pl.ds(base, HALF), :] = acc.astype(jnp.bfloat16)

        def xchg(step, rows_base, n_rows, partner, dst):
            rdma = pltpu.make_async_remote_copy(
                src_ref=out_ref.at[:, pl.ds(rows_base, n_rows), :],
                dst_ref=dst,
                send_sem=send_sems.at[step],
                recv_sem=recv_sems.at[step],
                device_id=(partner,),
                device_id_type=pl.DeviceIdType.MESH,
            )
            return rdma

        def add_into(rows_base, n_rows, rbuf):
            sl = (slice(None), pl.ds(rows_base, n_rows), slice(None))
            out_ref[sl] = (
                out_ref[sl].astype(jnp.float32)
                + rbuf[...].astype(jnp.float32)
            ).astype(jnp.bfloat16)

        b1_send = (1 - i0) * HALF
        b1_keep = i0 * HALF
        compute_half(b1_send)
        rdma1 = xchg(0, b1_send, HALF, p1, rbuf1)
        if _COMM:
            rdma1.start()
        compute_half(b1_keep)
        if _COMM:
            rdma1.wait()
        else:
            rbuf1[...] = out_ref[:, pl.ds(b1_send, HALF), :]
        add_into(b1_keep, HALF, rbuf1)

        b2_send = b1_keep + (1 - i2) * 128
        b2_keep = b1_keep + i2 * 128
        rdma2 = xchg(1, b2_send, 128, p4, rbuf2)
        if _COMM:
            rdma2.start()
            rdma2.wait()
        else:
            rbuf2[...] = out_ref[:, pl.ds(b2_send, 128), :]
        add_into(b2_keep, 128, rbuf2)

        b3_send = b2_keep + (1 - i1) * 64
        b3_keep = b2_keep + i1 * 64
        rdma3 = xchg(2, b3_send, 64, p2, rbuf3)
        if _COMM:
            rdma3.start()
            rdma3.wait()
        else:
            rbuf3[...] = out_ref[:, pl.ds(b3_send, 64), :]
        add_into(b3_keep, 64, rbuf3)

        if _COMM:
            for step, (rows_base, n_rows, partner) in enumerate(
                [(b3_keep, 64, p2), (b2_keep, 128, p4), (b1_keep, HALF, p1)]
            ):
                rdma = xchg(
                    3 + step, rows_base, n_rows, partner,
                    out_ref.at[:, pl.ds(rows_base, n_rows), :],
                )
                rdma.start()
                rdma.wait()

    return pl.pallas_call(
        body,
        out_shape=jax.ShapeDtypeStruct((B, SQ, DM), jnp.bfloat16),
        in_specs=[pl.BlockSpec(memory_space=pltpu.VMEM)] * 5,
        out_specs=pl.BlockSpec(memory_space=pltpu.VMEM),
        scratch_shapes=[
            pltpu.VMEM((B, HALF, DM), jnp.bfloat16),
            pltpu.VMEM((B, 128, DM), jnp.bfloat16),
            pltpu.VMEM((B, 64, DM), jnp.bfloat16),
            pltpu.SemaphoreType.DMA((6,)),
            pltpu.SemaphoreType.DMA((6,)),
        ],
        compiler_params=pltpu.CompilerParams(
            collective_id=0 if _COMM else None
        ),
    )(x_b, Wq_r, K, V, Wo_r)


# device time: 59157 ns/iter; 1.4798x vs baseline; 1.1254x over previous
import jax
import jax.numpy as jnp
from jax import lax
from jax.experimental import pallas as pl
from jax.experimental.pallas import tpu as pltpu

N_DEV = 8
B, SQ, SKV, HQ, DH, DM = 2, 512, 512, 64, 64, 768
H_PER = HQ // N_DEV
CHUNK = SQ // N_DEV
BLK = 2 * CHUNK
BAND = BLK + 2 * 128
WINDOW = 128
_COMM = True


def kernel(x, Wq, K_ext, V_ext, Wo):
    my = lax.axis_index("i")

    K = lax.dynamic_slice_in_dim(K_ext, my * H_PER, H_PER, axis=2)
    V = lax.dynamic_slice_in_dim(V_ext, my * H_PER, H_PER, axis=2)
    K = jnp.transpose(K, (2, 0, 1, 3)).astype(jnp.bfloat16)
    V = jnp.transpose(V, (2, 0, 1, 3)).astype(jnp.bfloat16)
    Wq_r = (jnp.transpose(Wq.reshape(DM, H_PER, DH), (1, 0, 2))
            * 0.125).astype(jnp.bfloat16)
    Wo_b = Wo.astype(jnp.bfloat16)
    x_b = x.astype(jnp.bfloat16)

    def body(x_ref, wq_ref, k_ref, v_ref, wo_ref, out_ref,
             rs_buf, ss_rs, ag_ss, rs_sem, ag_sem, dummy_sem):
        my_i = lax.axis_index("i")

        if _COMM:
            barrier = pltpu.get_barrier_semaphore()
            for k in range(1, N_DEV):
                pl.semaphore_signal(
                    barrier, inc=1,
                    device_id=((my_i + k) % N_DEV,),
                    device_id_type=pl.DeviceIdType.MESH,
                )
            pl.semaphore_wait(barrier, N_DEV - 1)

        rr = lax.broadcasted_iota(jnp.int32, (BLK, BAND), 0)
        cc = lax.broadcasted_iota(jnp.int32, (BLK, BAND), 1)

        def compute_block(rb):
            cb = pl.multiple_of(jnp.clip(rb - WINDOW, 0, SKV - BAND), 128)
            bias = jnp.where(
                jnp.abs((rr + rb) - (cc + cb)) <= WINDOW, 0.0, -1e9
            ).astype(jnp.float32)
            for b in range(B):
                xb = x_ref[b, pl.ds(rb, BLK), :]
                ctxs = []
                for h in range(H_PER):
                    q = jnp.dot(xb, wq_ref[h],
                                preferred_element_type=jnp.float32
                                ).astype(jnp.bfloat16)
                    ksl = k_ref[h, b, pl.ds(cb, BAND), :]
                    s = lax.dot_general(
                        q, ksl, (((1,), (1,)), ((), ())),
                        preferred_element_type=jnp.float32,
                    ) + bias
                    w = jnp.exp(s)
                    denom = jnp.sum(w, axis=-1, keepdims=True)
                    vsl = v_ref[h, b, pl.ds(cb, BAND), :]
                    ctx = jnp.dot(w.astype(jnp.bfloat16), vsl,
                                  preferred_element_type=jnp.float32)
                    ctxs.append((ctx / denom).astype(jnp.bfloat16))
                ctx_all = jnp.concatenate(ctxs, axis=1)
                o = jnp.dot(ctx_all, wo_ref[...],
                            preferred_element_type=jnp.float32)
                out_ref[b, pl.ds(rb, BLK), :] = o.astype(jnp.bfloat16)

        def send_chunk(dest, sem_slot, descs):
            slot = (my_i - dest) % N_DEV
            rdma = pltpu.make_async_remote_copy(
                src_ref=out_ref.at[:, pl.ds(dest * CHUNK, CHUNK), :],
                dst_ref=rs_buf.at[slot],
                send_sem=ss_rs.at[sem_slot],
                recv_sem=rs_sem,
                device_id=(dest,),
                device_id_type=pl.DeviceIdType.MESH,
            )
            rdma.start()
            descs.append(rdma)

        mb = my_i // 2
        rs_descs = []
        for t in range(4):
            blk = (mb + 1 + t) % 4
            compute_block(blk * BLK)
            if not _COMM:
                continue
            if t < 3:
                send_chunk(2 * blk, 2 * t, rs_descs)
                send_chunk(2 * blk + 1, 2 * t + 1, rs_descs)
            else:
                send_chunk(my_i ^ 1, 6, rs_descs)

        my_rows = (slice(None), pl.ds(my_i * CHUNK, CHUNK), slice(None))
        if _COMM:
            for k in range(1, N_DEV):
                recv = pltpu.make_async_remote_copy(
                    src_ref=rs_buf.at[k], dst_ref=rs_buf.at[k],
                    send_sem=dummy_sem, recv_sem=rs_sem,
                    device_id=(my_i,), device_id_type=pl.DeviceIdType.MESH,
                )
                recv.wait_recv()
            acc = out_ref[my_rows].astype(jnp.float32)
            for k in range(1, N_DEV):
                acc = acc + rs_buf[k].astype(jnp.float32)
            out_ref[my_rows] = acc.astype(jnp.bfloat16)

            ag_descs = []
            for k in range(1, N_DEV):
                rdma = pltpu.make_async_remote_copy(
                    src_ref=out_ref.at[my_rows],
                    dst_ref=out_ref.at[my_rows],
                    send_sem=ag_ss.at[k - 1],
                    recv_sem=ag_sem,
                    device_id=((my_i + k) % N_DEV,),
                    device_id_type=pl.DeviceIdType.MESH,
                )
                rdma.start()
                ag_descs.append(rdma)
            for r in rs_descs:
                r.wait_send()
            for k in range(1, N_DEV):
                recv = pltpu.make_async_remote_copy(
                    src_ref=out_ref.at[:, pl.ds(k * CHUNK, CHUNK), :],
                    dst_ref=out_ref.at[:, pl.ds(k * CHUNK, CHUNK), :],
                    send_sem=dummy_sem, recv_sem=ag_sem,
                    device_id=(my_i,), device_id_type=pl.DeviceIdType.MESH,
                )
                recv.wait_recv()
            for r in ag_descs:
                r.wait_send()

    return pl.pallas_call(
        body,
        out_shape=jax.ShapeDtypeStruct((B, SQ, DM), jnp.bfloat16),
        in_specs=[pl.BlockSpec(memory_space=pltpu.VMEM)] * 5,
        out_specs=pl.BlockSpec(memory_space=pltpu.VMEM),
        scratch_shapes=[
            pltpu.VMEM((N_DEV, B, CHUNK, DM), jnp.bfloat16),
            pltpu.SemaphoreType.DMA((7,)),
            pltpu.SemaphoreType.DMA((7,)),
            pltpu.SemaphoreType.DMA,
            pltpu.SemaphoreType.DMA,
            pltpu.SemaphoreType.DMA,
        ],
        compiler_params=pltpu.CompilerParams(
            collective_id=0 if _COMM else None
        ),
    )(x_b, Wq_r, K, V, Wo_b)


# device time: 46052 ns/iter; 1.9009x vs baseline; 1.2846x over previous
import jax
import jax.numpy as jnp
from jax import lax
from jax.experimental import pallas as pl
from jax.experimental.pallas import tpu as pltpu

N_DEV = 8
B, SQ, SKV, HQ, DH, DM = 2, 512, 512, 64, 64, 768
H_PER = HQ // N_DEV
CHUNK = SQ // N_DEV
BLK = 2 * CHUNK
BAND = BLK + 2 * 128
WINDOW = 128
_COMM = True


def kernel(x, Wq, K_ext, V_ext, Wo):
    my = lax.axis_index("i")

    K = lax.dynamic_slice_in_dim(K_ext, my * H_PER, H_PER, axis=2)
    V = lax.dynamic_slice_in_dim(V_ext, my * H_PER, H_PER, axis=2)
    K = jnp.transpose(K, (2, 0, 1, 3)).astype(jnp.bfloat16)
    V = jnp.transpose(V, (2, 0, 1, 3)).astype(jnp.bfloat16)
    Wq_r = (Wq * 0.125).astype(jnp.bfloat16)
    Wo_b = Wo.astype(jnp.bfloat16)
    x_b = x.astype(jnp.bfloat16)

    def body(x_ref, wq_ref, k_ref, v_ref, wo_ref, out_ref,
             rs_buf, ss_rs, ag_ss, rs_sem, ag_sem, dummy_sem):
        my_i = lax.axis_index("i")

        if _COMM:
            barrier = pltpu.get_barrier_semaphore()
            for k in range(1, N_DEV):
                pl.semaphore_signal(
                    barrier, inc=1,
                    device_id=((my_i + k) % N_DEV,),
                    device_id_type=pl.DeviceIdType.MESH,
                )
            pl.semaphore_wait(barrier, N_DEV - 1)

        rr = lax.broadcasted_iota(jnp.int32, (BLK, BAND), 0)
        cc = lax.broadcasted_iota(jnp.int32, (BLK, BAND), 1)

        def compute_block(rb):
            cb = pl.multiple_of(jnp.clip(rb - WINDOW, 0, SKV - BAND), 128)
            bias = jnp.where(
                jnp.abs((rr + rb) - (cc + cb)) <= WINDOW, 0.0, -1e9
            ).astype(jnp.float32)
            for b in range(B):
                xb = x_ref[b, pl.ds(rb, BLK), :]
                q_all = jnp.dot(xb, wq_ref[...],
                                preferred_element_type=jnp.float32
                                ).astype(jnp.bfloat16)
                ctxs = []
                for h in range(H_PER):
                    q = q_all[:, h * DH:(h + 1) * DH]
                    ksl = k_ref[h, b, pl.ds(cb, BAND), :]
                    s = lax.dot_general(
                        q, ksl, (((1,), (1,)), ((), ())),
                        preferred_element_type=jnp.float32,
                    ) + bias
                    w = jnp.exp(s)
                    denom = jnp.sum(w, axis=-1, keepdims=True)
                    vsl = v_ref[h, b, pl.ds(cb, BAND), :]
                    ctx = jnp.dot(w.astype(jnp.bfloat16), vsl,
                                  preferred_element_type=jnp.float32)
                    ctxs.append((ctx / denom).astype(jnp.bfloat16))
                ctx_all = jnp.concatenate(ctxs, axis=1)
                o = jnp.dot(ctx_all, wo_ref[...],
                            preferred_element_type=jnp.float32)
                out_ref[b, pl.ds(rb, BLK), :] = o.astype(jnp.bfloat16)

        def send_chunk(dest, sem_slot, descs):
            slot = (my_i - dest) % N_DEV
            rdma = pltpu.make_async_remote_copy(
                src_ref=out_ref.at[:, pl.ds(dest * CHUNK, CHUNK), :],
                dst_ref=rs_buf.at[slot],
                send_sem=ss_rs.at[sem_slot],
                recv_sem=rs_sem,
                device_id=(dest,),
                device_id_type=pl.DeviceIdType.MESH,
            )
            rdma.start()
            descs.append(rdma)

        mb = my_i // 2
        rs_descs = []
        for t in range(4):
            blk = (mb + 1 + t) % 4
            compute_block(blk * BLK)
            if not _COMM:
                continue
            if t < 3:
                send_chunk(2 * blk, 2 * t, rs_descs)
                send_chunk(2 * blk + 1, 2 * t + 1, rs_descs)
            else:
                send_chunk(my_i ^ 1, 6, rs_descs)

        my_rows = (slice(None), pl.ds(my_i * CHUNK, CHUNK), slice(None))
        if _COMM:
            for k in range(1, N_DEV):
                recv = pltpu.make_async_remote_copy(
                    src_ref=rs_buf.at[k], dst_ref=rs_buf.at[k],
                    send_sem=dummy_sem, recv_sem=rs_sem,
                    device_id=(my_i,), device_id_type=pl.DeviceIdType.MESH,
                )
                recv.wait_recv()
            acc = out_ref[my_rows].astype(jnp.float32)
            for k in range(1, N_DEV):
                acc = acc + rs_buf[k].astype(jnp.float32)
            out_ref[my_rows] = acc.astype(jnp.bfloat16)

            ag_descs = []
            for k in range(1, N_DEV):
                rdma = pltpu.make_async_remote_copy(
                    src_ref=out_ref.at[my_rows],
                    dst_ref=out_ref.at[my_rows],
                    send_sem=ag_ss.at[k - 1],
                    recv_sem=ag_sem,
                    device_id=((my_i + k) % N_DEV,),
                    device_id_type=pl.DeviceIdType.MESH,
                )
                rdma.start()
                ag_descs.append(rdma)
            for r in rs_descs:
                r.wait_send()
            for k in range(1, N_DEV):
                recv = pltpu.make_async_remote_copy(
                    src_ref=out_ref.at[:, pl.ds(k * CHUNK, CHUNK), :],
                    dst_ref=out_ref.at[:, pl.ds(k * CHUNK, CHUNK), :],
                    send_sem=dummy_sem, recv_sem=ag_sem,
                    device_id=(my_i,), device_id_type=pl.DeviceIdType.MESH,
                )
                recv.wait_recv()
            for r in ag_descs:
                r.wait_send()

    return pl.pallas_call(
        body,
        out_shape=jax.ShapeDtypeStruct((B, SQ, DM), jnp.bfloat16),
        in_specs=[pl.BlockSpec(memory_space=pltpu.VMEM)] * 5,
        out_specs=pl.BlockSpec(memory_space=pltpu.VMEM),
        scratch_shapes=[
            pltpu.VMEM((N_DEV, B, CHUNK, DM), jnp.bfloat16),
            pltpu.SemaphoreType.DMA((7,)),
            pltpu.SemaphoreType.DMA((7,)),
            pltpu.SemaphoreType.DMA,
            pltpu.SemaphoreType.DMA,
            pltpu.SemaphoreType.DMA,
        ],
        compiler_params=pltpu.CompilerParams(
            collective_id=0 if _COMM else None
        ),
    )(x_b, Wq_r, K, V, Wo_b)
